# Initial kernel scaffold; baseline (speedup 1.0000x reference)
import jax
import jax.numpy as jnp
from jax.experimental import pallas as pl


def kernel(uav_embeddings, target_embeddings, distances, Wq, Wk, Wv, Wo, bo):
    raise NotImplementedError("write your pallas kernel here")


# speedup vs baseline: 11.0283x; 11.0283x over previous
import functools

import numpy as np
import jax
import jax.numpy as jnp
from jax import lax
from jax.experimental import pallas as pl
from jax.experimental.pallas import tpu as pltpu
from jax.experimental.pallas import tpu_sc as plsc

B, U, T, D, H = 4, 2048, 4096, 256, 8
DH = D // H
K = 16
SCALE = 1.0 / np.sqrt(DH)

UB = 256


def _topk_body(dist_ref, idx_ref, vals_ref):
    b = pl.program_id(0)
    vals_ref[...] = dist_ref[0]
    iota = lax.broadcasted_iota(jnp.int32, (UB, T), 1)
    cols = []
    for _ in range(K):
        v = vals_ref[...]
        m = jnp.min(v, axis=1, keepdims=True)
        cand = jnp.where(v == m, iota, T)
        j = jnp.min(cand, axis=1, keepdims=True)
        vals_ref[...] = jnp.where(iota == j, jnp.float32(np.inf), v)
        cols.append(j)
    idx_ref[0] = jnp.concatenate(cols, axis=1) + b * T


def _topk(distances):
    return pl.pallas_call(
        _topk_body,
        grid=(B, U // UB),
        in_specs=[pl.BlockSpec((1, UB, T), lambda b, u: (b, u, 0))],
        out_specs=pl.BlockSpec((1, UB, K), lambda b, u: (b, u, 0)),
        out_shape=jax.ShapeDtypeStruct((B, U, K), jnp.int32),
        scratch_shapes=[pltpu.VMEM((UB, T), jnp.float32)],
    )(distances)


_NC, _NS = 2, 16
_NW = _NC * _NS
_NROWS = B * U * K
_RPW = _NROWS // _NW
_CH = 32
_NCHUNK = _RPW // _CH


def _gather_body(table_hbm, idx_hbm, out_hbm, idx_v, rows_v, sem):
    wid = lax.axis_index("s") * _NC + lax.axis_index("c")
    base = wid * _RPW

    def chunk(c, carry):
        off = pl.multiple_of(base + c * _CH, _CH)
        pltpu.sync_copy(idx_hbm.at[pl.ds(off, _CH)], idx_v)
        pltpu.async_copy(table_hbm.at[idx_v], rows_v, sem).wait()
        pltpu.sync_copy(rows_v, out_hbm.at[pl.ds(off, _CH)])
        return carry

    lax.fori_loop(0, _NCHUNK, chunk, 0)


@functools.partial(
    pl.kernel,
    out_type=jax.ShapeDtypeStruct((_NROWS, D), jnp.float32),
    mesh=plsc.VectorSubcoreMesh(core_axis_name="c", subcore_axis_name="s"),
    scratch_types=[
        pltpu.VMEM((_CH,), jnp.int32),
        pltpu.VMEM((_CH, D), jnp.float32),
        pltpu.SemaphoreType.DMA,
    ],
)
def _gather(table_hbm, idx_hbm, out_hbm, idx_v, rows_v, sem):
    _gather_body(table_hbm, idx_hbm, out_hbm, idx_v, rows_v, sem)


QB = 128


def _attn_body(uav_ref, sel_ref, wq_ref, wk_ref, wv_ref, wo_ref, bo_ref, out_ref):
    f32 = jnp.float32
    bf16 = jnp.bfloat16
    dims_t = (((1,), (1,)), ((), ()))

    x = uav_ref[...]
    q = lax.dot_general(x, wq_ref[...], dims_t, preferred_element_type=f32)
    sel = sel_ref[...].astype(bf16)
    kmat = lax.dot_general(sel, wk_ref[...].astype(bf16), dims_t,
                           preferred_element_type=f32)
    vmat = lax.dot_general(sel, wv_ref[...].astype(bf16), dims_t,
                           preferred_element_type=f32)

    dd = lax.broadcasted_iota(jnp.int32, (D, H), 0)
    hh = lax.broadcasted_iota(jnp.int32, (D, H), 1)
    G = (dd // DH == hh).astype(f32)

    qe = jnp.broadcast_to(q[:, None, :], (QB, K, D)).reshape(QB * K, D)
    prod = qe * kmat
    scores = lax.dot_general(prod, G, (((1,), (0,)), ((), ())),
                             preferred_element_type=f32) * SCALE

    s = scores.reshape(QB, K, H)
    m = jnp.max(s, axis=1, keepdims=True)
    e = jnp.exp(s - m)
    p = (e / jnp.sum(e, axis=1, keepdims=True)).reshape(QB * K, H)

    pfull = lax.dot_general(p, G, (((1,), (1,)), ((), ())),
                            preferred_element_type=f32)
    ctx = (pfull * vmat).reshape(QB, K, D)
    attn_out = jnp.sum(ctx, axis=1)

    out = lax.dot_general(attn_out, wo_ref[...], dims_t,
                          preferred_element_type=f32) + bo_ref[...]
    out_ref[...] = out


def _attention(uav_flat, sel, Wq, Wk, Wv, Wo, bo2):
    nq = B * U
    wspec = pl.BlockSpec((D, D), lambda i: (0, 0))
    return pl.pallas_call(
        _attn_body,
        grid=(nq // QB,),
        in_specs=[
            pl.BlockSpec((QB, D), lambda i: (i, 0)),
            pl.BlockSpec((QB * K, D), lambda i: (i, 0)),
            wspec, wspec, wspec, wspec,
            pl.BlockSpec((1, D), lambda i: (0, 0)),
        ],
        out_specs=pl.BlockSpec((QB, D), lambda i: (i, 0)),
        out_shape=jax.ShapeDtypeStruct((nq, D), jnp.float32),
    )(uav_flat, sel, Wq, Wk, Wv, Wo, bo2)


def kernel(uav_embeddings, target_embeddings, distances, Wq, Wk, Wv, Wo, bo):
    idx = _topk(distances)
    sel = _gather(target_embeddings.reshape(B * T, D), idx.reshape(_NROWS))
    out = _attention(uav_embeddings.reshape(B * U, D), sel,
                     Wq, Wk, Wv, Wo, bo.reshape(1, D))
    return out.reshape(B, U, D)


# speedup vs baseline: 12.7354x; 1.1548x over previous
import functools

import numpy as np
import jax
import jax.numpy as jnp
from jax import lax
from jax.experimental import pallas as pl
from jax.experimental.pallas import tpu as pltpu
from jax.experimental.pallas import tpu_sc as plsc

B, U, T, D, H = 4, 2048, 4096, 256, 8
DH = D // H
K = 16
SCALE = 1.0 / np.sqrt(DH)

UB = 256


def _topk_body(dist_ref, idx_ref, vals_ref):
    b = pl.program_id(0)
    vals_ref[...] = dist_ref[0]
    iota = lax.broadcasted_iota(jnp.int32, (UB, T), 1)
    cols = []
    for _ in range(K):
        v = vals_ref[...]
        m = jnp.min(v, axis=1, keepdims=True)
        cand = jnp.where(v == m, iota, T)
        j = jnp.min(cand, axis=1, keepdims=True)
        vals_ref[...] = jnp.where(iota == j, jnp.float32(np.inf), v)
        cols.append(j)
    idx_ref[0] = jnp.concatenate(cols, axis=1) + b * T


def _topk(distances):
    return pl.pallas_call(
        _topk_body,
        grid=(B, U // UB),
        in_specs=[pl.BlockSpec((1, UB, T), lambda b, u: (b, u, 0))],
        out_specs=pl.BlockSpec((1, UB, K), lambda b, u: (b, u, 0)),
        out_shape=jax.ShapeDtypeStruct((B, U, K), jnp.int32),
        scratch_shapes=[pltpu.VMEM((UB, T), jnp.float32)],
    )(distances)


_NC, _NS = 2, 16
_NW = _NC * _NS
_NROWS = B * U * K
_RPW = _NROWS // _NW
_CH = 64
_NCHUNK = _RPW // _CH


def _gather_body(table_hbm, idx_hbm, out_hbm, idx0, idx1, rows0, rows1,
                 isem0, isem1, gsem0, gsem1, osem0, osem1):
    wid = lax.axis_index("s") * _NC + lax.axis_index("c")
    base = wid * _RPW
    idxs, rows = (idx0, idx1), (rows0, rows1)
    isems, gsems, osems = (isem0, isem1), (gsem0, gsem1), (osem0, osem1)

    def off_of(c):
        return pl.multiple_of(base + c * _CH, _CH)

    pltpu.async_copy(idx_hbm.at[pl.ds(off_of(0), _CH)], idxs[0], isems[0])
    pltpu.async_copy(idx_hbm.at[pl.ds(off_of(1), _CH)], idxs[1], isems[1])

    def pair(g, carry):
        for b in range(2):
            c = g * 2 + b
            off = off_of(c)

            @pl.when(g > 0)
            def _():
                pltpu.make_async_copy(
                    rows[b], out_hbm.at[pl.ds(off_of(c - 2), _CH)], osems[b]
                ).wait()

            pltpu.make_async_copy(
                idx_hbm.at[pl.ds(off, _CH)], idxs[b], isems[b]).wait()
            pltpu.async_copy(table_hbm.at[idxs[b]], rows[b], gsems[b])
            pltpu.make_async_copy(table_hbm.at[idxs[b]], rows[b], gsems[b]).wait()

            @pl.when(c + 2 < _NCHUNK)
            def _():
                pltpu.async_copy(
                    idx_hbm.at[pl.ds(off_of(c + 2), _CH)], idxs[b], isems[b])

            pltpu.async_copy(rows[b], out_hbm.at[pl.ds(off, _CH)], osems[b])
        return carry

    lax.fori_loop(0, _NCHUNK // 2, pair, 0)
    for b in range(2):
        pltpu.make_async_copy(
            rows[b],
            out_hbm.at[pl.ds(off_of(_NCHUNK - 2 + b), _CH)],
            osems[b],
        ).wait()


@functools.partial(
    pl.kernel,
    out_type=jax.ShapeDtypeStruct((_NROWS, D), jnp.float32),
    mesh=plsc.VectorSubcoreMesh(core_axis_name="c", subcore_axis_name="s"),
    scratch_types=[
        pltpu.VMEM((_CH,), jnp.int32),
        pltpu.VMEM((_CH,), jnp.int32),
        pltpu.VMEM((_CH, D), jnp.float32),
        pltpu.VMEM((_CH, D), jnp.float32),
        pltpu.SemaphoreType.DMA,
        pltpu.SemaphoreType.DMA,
        pltpu.SemaphoreType.DMA,
        pltpu.SemaphoreType.DMA,
        pltpu.SemaphoreType.DMA,
        pltpu.SemaphoreType.DMA,
    ],
)
def _gather(table_hbm, idx_hbm, out_hbm, idx0, idx1, rows0, rows1,
            isem0, isem1, gsem0, gsem1, osem0, osem1):
    _gather_body(table_hbm, idx_hbm, out_hbm, idx0, idx1, rows0, rows1,
                 isem0, isem1, gsem0, gsem1, osem0, osem1)


QB = 128


def _attn_body(uav_ref, sel_ref, wq_ref, wk_ref, wv_ref, wo_ref, bo_ref, out_ref):
    f32 = jnp.float32
    bf16 = jnp.bfloat16
    dims_t = (((1,), (1,)), ((), ()))

    x = uav_ref[...]
    q = lax.dot_general(x, wq_ref[...], dims_t, preferred_element_type=f32)
    sel = sel_ref[...].astype(bf16)
    kmat = lax.dot_general(sel, wk_ref[...].astype(bf16), dims_t,
                           preferred_element_type=f32)
    vmat = lax.dot_general(sel, wv_ref[...].astype(bf16), dims_t,
                           preferred_element_type=f32)

    dd = lax.broadcasted_iota(jnp.int32, (D, H), 0)
    hh = lax.broadcasted_iota(jnp.int32, (D, H), 1)
    G = (dd // DH == hh).astype(f32)

    qe = jnp.broadcast_to(q[:, None, :], (QB, K, D)).reshape(QB * K, D)
    prod = qe * kmat
    scores = lax.dot_general(prod, G, (((1,), (0,)), ((), ())),
                             preferred_element_type=f32) * SCALE

    s = scores.reshape(QB, K, H)
    m = jnp.max(s, axis=1, keepdims=True)
    e = jnp.exp(s - m)
    p = (e / jnp.sum(e, axis=1, keepdims=True)).reshape(QB * K, H)

    pfull = lax.dot_general(p, G, (((1,), (1,)), ((), ())),
                            preferred_element_type=f32)
    ctx = (pfull * vmat).reshape(QB, K, D)
    attn_out = jnp.sum(ctx, axis=1)

    out = lax.dot_general(attn_out, wo_ref[...], dims_t,
                          preferred_element_type=f32) + bo_ref[...]
    out_ref[...] = out


def _attention(uav_flat, sel, Wq, Wk, Wv, Wo, bo2):
    nq = B * U
    wspec = pl.BlockSpec((D, D), lambda i: (0, 0))
    return pl.pallas_call(
        _attn_body,
        grid=(nq // QB,),
        in_specs=[
            pl.BlockSpec((QB, D), lambda i: (i, 0)),
            pl.BlockSpec((QB * K, D), lambda i: (i, 0)),
            wspec, wspec, wspec, wspec,
            pl.BlockSpec((1, D), lambda i: (0, 0)),
        ],
        out_specs=pl.BlockSpec((QB, D), lambda i: (i, 0)),
        out_shape=jax.ShapeDtypeStruct((nq, D), jnp.float32),
    )(uav_flat, sel, Wq, Wk, Wv, Wo, bo2)


def kernel(uav_embeddings, target_embeddings, distances, Wq, Wk, Wv, Wo, bo):
    idx = _topk(distances)
    sel = _gather(target_embeddings.reshape(B * T, D), idx.reshape(_NROWS))
    out = _attention(uav_embeddings.reshape(B * U, D), sel,
                     Wq, Wk, Wv, Wo, bo.reshape(1, D))
    return out.reshape(B, U, D)
